# baseline (speedup 1.0000x reference)
import jax
import jax.numpy as jnp
from jax.experimental import pallas as pl
from jax.experimental.pallas import tpu as pltpu
from jax.experimental.pallas import tpu_sc as plsc


GATHER_WINDOW = 128


def _sc_gather(table, indices):
    b = indices.shape[1]
    d = table.shape[1]
    mesh = plsc.VectorSubcoreMesh(core_axis_name="core", subcore_axis_name="subcore")

    @pl.kernel(
        out_type=jax.ShapeDtypeStruct((b, d), table.dtype),
        mesh=mesh,
        scratch_types=[pltpu.SemaphoreType.DMA],
    )
    def kern(x_hbm, i_hbm, o_hbm, sem):
        def body(i_vmem, o_vmem):
            @pl.loop(0, GATHER_WINDOW)
            def _issue(j):
                row = i_vmem[0, pl.ds(j, 1)][0]
                pltpu.make_async_copy(x_hbm.at[row], o_vmem.at[j], sem).start()

            @pl.loop(0, GATHER_WINDOW)
            def _wait(j):
                row = i_vmem[0, pl.ds(j, 1)][0]
                pltpu.make_async_copy(x_hbm.at[row], o_vmem.at[j], sem).wait()

        pltpu.emit_pipeline(
            body,
            grid=(b // GATHER_WINDOW,),
            in_specs=[pl.BlockSpec((1, GATHER_WINDOW), index_map=lambda i: (0, i))],
            out_specs=[pl.BlockSpec((GATHER_WINDOW, d), index_map=lambda i: (i, 0))],
            core_axis_name=("core", "subcore"),
            dimension_semantics=(pltpu.PARALLEL,),
        )(i_hbm, o_hbm)

    return kern(table, indices)


def kernel(feat_static, n_timesteps, feat_time, embedding_weight):
    idx = jnp.squeeze(feat_static.astype(jnp.int32), axis=-1).reshape(1, -1)
    table_p = jnp.pad(embedding_weight, ((0, 8), (0, 0)))
    emb = _sc_gather(table_p, idx)
    t = feat_time.shape[1]
    rep = jnp.broadcast_to(emb[:, None, :], (emb.shape[0], t, emb.shape[1]))
    return jnp.concatenate([rep, feat_time], axis=-1)


# speedup vs baseline: 1.7241x; 1.7241x over previous
import jax
import jax.numpy as jnp
from jax.experimental import pallas as pl
from jax.experimental.pallas import tpu as pltpu
from jax.experimental.pallas import tpu_sc as plsc


GATHER_WINDOW = 128


def _sc_gather(table, indices):
    b = indices.shape[1]
    d = table.shape[1]
    mesh = plsc.VectorSubcoreMesh(core_axis_name="core", subcore_axis_name="subcore")

    @pl.kernel(
        out_type=jax.ShapeDtypeStruct((b, d), table.dtype),
        mesh=mesh,
        scratch_types=[pltpu.SemaphoreType.DMA],
    )
    def kern(x_hbm, i_hbm, o_hbm, sem):
        def body(i_vmem, o_vmem):
            @pl.loop(0, GATHER_WINDOW)
            def _issue(j):
                row = i_vmem[0, pl.ds(j, 1)][0]
                pltpu.make_async_copy(x_hbm.at[row], o_vmem.at[j], sem).start()

            @pl.loop(0, GATHER_WINDOW)
            def _wait(j):
                row = i_vmem[0, pl.ds(j, 1)][0]
                pltpu.make_async_copy(x_hbm.at[row], o_vmem.at[j], sem).wait()

        pltpu.emit_pipeline(
            body,
            grid=(b // GATHER_WINDOW,),
            in_specs=[pl.BlockSpec((1, GATHER_WINDOW), index_map=lambda i: (0, i))],
            out_specs=[pl.BlockSpec((GATHER_WINDOW, d), index_map=lambda i: (i, 0))],
            core_axis_name=("core", "subcore"),
            dimension_semantics=(pltpu.PARALLEL,),
        )(i_hbm, o_hbm)

    return kern(table, indices)


def kernel(feat_static, n_timesteps, feat_time, embedding_weight):
    idx = jnp.squeeze(feat_static.astype(jnp.int32), axis=-1).reshape(1, -1)
    emb = _sc_gather(embedding_weight, idx)
    t = feat_time.shape[1]
    d = emb.shape[1]
    f = feat_time.shape[2]
    mask = jax.lax.broadcasted_iota(jnp.int32, (1, 1, d + f), 2) < d
    emb68 = jnp.pad(emb, ((0, 0), (0, f)))
    ft68 = jnp.pad(feat_time, ((0, 0), (0, 0), (d, 0)))
    return jnp.where(mask, emb68[:, None, :], ft68)
